# baseline (speedup 1.0000x reference)
import functools

import jax
import jax.numpy as jnp
from jax import lax
from jax.experimental import pallas as pl
from jax.experimental.pallas import tpu as pltpu
from jax.experimental.pallas import tpu_sc as plsc

D_IN = 64
D_H = 128
B_TOK = 16384 * 20
NW = 32
BPW = B_TOK // NW
CH = 512
NCHUNK = BPW // CH

CB = 512
N_TBLK = 977
HALF = N_TBLK * CB
V_PACK = 2 * HALF


def _tc_transpose(tableT):
  def body(a_ref, b_ref, o_ref):
    o_ref[:, 0:D_IN] = a_ref[...].T
    o_ref[:, D_IN:] = b_ref[...].T

  return pl.pallas_call(
      body,
      grid=(N_TBLK,),
      in_specs=[
          pl.BlockSpec((D_IN, CB), lambda i: (0, i)),
          pl.BlockSpec((D_IN, CB), lambda i: (0, i + N_TBLK)),
      ],
      out_specs=pl.BlockSpec((CB, 2 * D_IN), lambda i: (i, 0)),
      out_shape=jax.ShapeDtypeStruct((HALF, 2 * D_IN), jnp.float32),
  )(tableT, tableT)


def _sc_gather(table_rm, idx3):
  mesh = plsc.VectorSubcoreMesh(core_axis_name="c", subcore_axis_name="s")

  @functools.partial(
      pl.kernel,
      mesh=mesh,
      compiler_params=pltpu.CompilerParams(use_tc_tiling_on_sc=False),
      out_type=jax.ShapeDtypeStruct((B_TOK, D_IN), jnp.float32),
      scratch_types=[
          pltpu.VMEM((NCHUNK, CH), jnp.int32),
          pltpu.VMEM((CH, D_IN), jnp.float32),
          pltpu.VMEM((CH, D_IN), jnp.float32),
          pltpu.SemaphoreType.DMA,
          pltpu.SemaphoreType.DMA,
      ],
  )
  def k(table_hbm, idx_hbm, out_hbm, idx_v, buf0, buf1, sem0, sem1):
    wid = lax.axis_index("s") * 2 + lax.axis_index("c")
    base = wid * BPW
    pltpu.sync_copy(idx_hbm.at[wid], idx_v)
    bufs = (buf0, buf1)
    sems = (sem0, sem1)
    cps = [None, None]
    cps[0] = pltpu.async_copy(table_hbm.at[idx_v.at[0]], buf0, sem0)
    for c in range(NCHUNK):
      nxt = c + 1
      if nxt < NCHUNK:
        cps[nxt % 2] = pltpu.async_copy(
            table_hbm.at[idx_v.at[nxt]], bufs[nxt % 2], sems[nxt % 2])
      cps[c % 2].wait()
      pltpu.sync_copy(bufs[c % 2], out_hbm.at[pl.ds(base + c * CH, CH)])

  return k(table_rm, idx3)


def _tc_mlp(emb2, W1b, b1b, W2b, b2b):
  BLK = 2048
  n_rows = emb2.shape[0]

  def body(e_ref, w1_ref, b1_ref, w2_ref, b2_ref, o_ref):
    e = e_ref[...]
    h = jnp.dot(e, w1_ref[...], preferred_element_type=jnp.float32) + b1_ref[...]
    h = h * jax.nn.sigmoid(h)
    o_ref[...] = (
        jnp.dot(h, w2_ref[...], preferred_element_type=jnp.float32) + b2_ref[...]
    )

  return pl.pallas_call(
      body,
      grid=(n_rows // BLK,),
      in_specs=[
          pl.BlockSpec((BLK, 2 * D_IN), lambda i: (i, 0)),
          pl.BlockSpec((2 * D_IN, 2 * D_H), lambda i: (0, 0)),
          pl.BlockSpec((1, 2 * D_H), lambda i: (0, 0)),
          pl.BlockSpec((2 * D_H, 2 * D_H), lambda i: (0, 0)),
          pl.BlockSpec((1, 2 * D_H), lambda i: (0, 0)),
      ],
      out_specs=pl.BlockSpec((BLK, 2 * D_H), lambda i: (i, 0)),
      out_shape=jax.ShapeDtypeStruct((n_rows, 2 * D_H), jnp.float32),
  )(emb2, W1b, b1b, W2b, b2b)


def kernel(t, table, W1, b1, W2, b2):
  Bt, L = t.shape
  packed = _tc_transpose(table.T)
  table_rm = packed.reshape(V_PACK, D_IN)
  tq = jnp.where(t < HALF, 2 * t, 2 * (t - HALF) + 1)
  idx3 = tq.T.reshape(NW, NCHUNK, CH)
  emb = _sc_gather(table_rm, idx3)
  emb2 = emb.reshape(B_TOK // 2, 2 * D_IN)
  Z = jnp.zeros_like(W1)
  W1b = jnp.block([[W1, Z], [Z, W1]])
  Zh = jnp.zeros_like(W2)
  W2b = jnp.block([[W2, Zh], [Zh, W2]])
  b1b = jnp.concatenate([b1, b1]).reshape(1, 2 * D_H)
  b2b = jnp.concatenate([b2, b2]).reshape(1, 2 * D_H)
  out2 = _tc_mlp(emb2, W1b, b1b, W2b, b2b)
  return out2.reshape(L, Bt, D_H).transpose(1, 0, 2)


# speedup vs baseline: 1.5324x; 1.5324x over previous
import functools

import jax
import jax.numpy as jnp
from jax import lax
from jax.experimental import pallas as pl
from jax.experimental.pallas import tpu as pltpu
from jax.experimental.pallas import tpu_sc as plsc

D_IN = 64
D_H = 128
B_TOK = 16384 * 20
NW = 32
BPW = B_TOK // NW
CH = 512
NCHUNK = BPW // CH

CB = 2048
SUB = 512
N_SUPER = 245
NP = N_SUPER * CB
V_PACK = 2 * NP
LAST_B_BLK = 487


def _tc_transpose(tableT):
  def body(a_ref, b_ref, o_ref):
    for j in range(CB // SUB):
      sl = pl.ds(j * SUB, SUB)
      o_ref[sl, 0:D_IN] = a_ref[:, sl].T
      o_ref[sl, D_IN:] = b_ref[:, sl].T

  return pl.pallas_call(
      body,
      grid=(N_SUPER,),
      in_specs=[
          pl.BlockSpec((D_IN, CB), lambda i: (0, 2 * i)),
          pl.BlockSpec((D_IN, CB), lambda i: (0, jnp.minimum(2 * i + 1, LAST_B_BLK))),
      ],
      out_specs=pl.BlockSpec((CB, 2 * D_IN), lambda i: (i, 0)),
      out_shape=jax.ShapeDtypeStruct((NP, 2 * D_IN), jnp.float32),
  )(tableT, tableT)


def _sc_gather(table_rm, idx3):
  mesh = plsc.VectorSubcoreMesh(core_axis_name="c", subcore_axis_name="s")

  @functools.partial(
      pl.kernel,
      mesh=mesh,
      compiler_params=pltpu.CompilerParams(use_tc_tiling_on_sc=False),
      out_type=jax.ShapeDtypeStruct((B_TOK, D_IN), jnp.float32),
      scratch_types=[
          pltpu.VMEM((NCHUNK, CH), jnp.int32),
          pltpu.VMEM((CH, D_IN), jnp.float32),
          pltpu.VMEM((CH, D_IN), jnp.float32),
          pltpu.SemaphoreType.DMA,
          pltpu.SemaphoreType.DMA,
      ],
  )
  def k(table_hbm, idx_hbm, out_hbm, idx_v, buf0, buf1, sem0, sem1):
    wid = lax.axis_index("s") * 2 + lax.axis_index("c")
    base = wid * BPW
    pltpu.sync_copy(idx_hbm.at[wid], idx_v)
    bufs = (buf0, buf1)
    sems = (sem0, sem1)
    cps = [None, None]
    cps[0] = pltpu.async_copy(table_hbm.at[idx_v.at[0]], buf0, sem0)
    for c in range(NCHUNK):
      nxt = c + 1
      if nxt < NCHUNK:
        cps[nxt % 2] = pltpu.async_copy(
            table_hbm.at[idx_v.at[nxt]], bufs[nxt % 2], sems[nxt % 2])
      cps[c % 2].wait()
      pltpu.sync_copy(bufs[c % 2], out_hbm.at[pl.ds(base + c * CH, CH)])

  return k(table_rm, idx3)


def _tc_mlp(emb2, W1b, b1b, W2b, b2b):
  BLK = 2048
  n_rows = emb2.shape[0]

  def body(e_ref, w1_ref, b1_ref, w2_ref, b2_ref, o_ref):
    e = e_ref[...]
    h = jnp.dot(e, w1_ref[...], preferred_element_type=jnp.float32) + b1_ref[...]
    h = h * jax.nn.sigmoid(h)
    o_ref[...] = (
        jnp.dot(h, w2_ref[...], preferred_element_type=jnp.float32) + b2_ref[...]
    )

  return pl.pallas_call(
      body,
      grid=(n_rows // BLK,),
      in_specs=[
          pl.BlockSpec((BLK, 2 * D_IN), lambda i: (i, 0)),
          pl.BlockSpec((2 * D_IN, 2 * D_H), lambda i: (0, 0)),
          pl.BlockSpec((1, 2 * D_H), lambda i: (0, 0)),
          pl.BlockSpec((2 * D_H, 2 * D_H), lambda i: (0, 0)),
          pl.BlockSpec((1, 2 * D_H), lambda i: (0, 0)),
      ],
      out_specs=pl.BlockSpec((BLK, 2 * D_H), lambda i: (i, 0)),
      out_shape=jax.ShapeDtypeStruct((n_rows, 2 * D_H), jnp.float32),
  )(emb2, W1b, b1b, W2b, b2b)


def kernel(t, table, W1, b1, W2, b2):
  Bt, L = t.shape
  packed = _tc_transpose(table.T)
  table_rm = packed.reshape(V_PACK, D_IN)
  tq = 2 * ((t // (2 * CB)) * CB + t % CB) + (t // CB) % 2
  idx3 = tq.T.reshape(NW, NCHUNK, CH)
  emb = _sc_gather(table_rm, idx3)
  emb2 = emb.reshape(B_TOK // 2, 2 * D_IN)
  Z = jnp.zeros_like(W1)
  W1b = jnp.block([[W1, Z], [Z, W1]])
  Zh = jnp.zeros_like(W2)
  W2b = jnp.block([[W2, Zh], [Zh, W2]])
  b1b = jnp.concatenate([b1, b1]).reshape(1, 2 * D_H)
  b2b = jnp.concatenate([b2, b2]).reshape(1, 2 * D_H)
  out2 = _tc_mlp(emb2, W1b, b1b, W2b, b2b)
  return out2.reshape(L, Bt, D_H).transpose(1, 0, 2)


# speedup vs baseline: 1.6987x; 1.1085x over previous
import functools

import jax
import jax.numpy as jnp
from jax import lax
from jax.experimental import pallas as pl
from jax.experimental.pallas import tpu as pltpu
from jax.experimental.pallas import tpu_sc as plsc

D_IN = 64
D_H = 128
B_TOK = 16384 * 20
NW = 32
BPW = B_TOK // NW
CH = 512
NCHUNK = BPW // CH

CB = 4096
SUB = 512
N_SUPER = 123
NP = N_SUPER * CB
V_PACK = 2 * NP
LAST_B_BLK = 244


def _tc_transpose(tableT):
  def body(a_ref, b_ref, o_ref):
    for j in range(CB // SUB):
      sl = pl.ds(j * SUB, SUB)
      o_ref[sl, 0:D_IN] = a_ref[:, sl].T
      o_ref[sl, D_IN:] = b_ref[:, sl].T

  return pl.pallas_call(
      body,
      grid=(N_SUPER,),
      in_specs=[
          pl.BlockSpec((D_IN, CB), lambda i: (0, 2 * i)),
          pl.BlockSpec((D_IN, CB), lambda i: (0, jnp.minimum(2 * i + 1, LAST_B_BLK))),
      ],
      out_specs=pl.BlockSpec((CB, 2 * D_IN), lambda i: (i, 0)),
      out_shape=jax.ShapeDtypeStruct((NP, 2 * D_IN), jnp.float32),
  )(tableT, tableT)


def _sc_gather(table_rm, idx3):
  mesh = plsc.VectorSubcoreMesh(core_axis_name="c", subcore_axis_name="s")

  @functools.partial(
      pl.kernel,
      mesh=mesh,
      compiler_params=pltpu.CompilerParams(use_tc_tiling_on_sc=False),
      out_type=jax.ShapeDtypeStruct((B_TOK, D_IN), jnp.float32),
      scratch_types=[
          pltpu.VMEM((NCHUNK, CH), jnp.int32),
          pltpu.VMEM((CH, D_IN), jnp.float32),
          pltpu.VMEM((CH, D_IN), jnp.float32),
          pltpu.SemaphoreType.DMA,
          pltpu.SemaphoreType.DMA,
      ],
  )
  def k(table_hbm, idx_hbm, out_hbm, idx_v, buf0, buf1, sem0, sem1):
    wid = lax.axis_index("s") * 2 + lax.axis_index("c")
    base = wid * BPW
    pltpu.sync_copy(idx_hbm.at[wid], idx_v)
    bufs = (buf0, buf1)
    sems = (sem0, sem1)
    cps = [None, None]
    cps[0] = pltpu.async_copy(table_hbm.at[idx_v.at[0]], buf0, sem0)
    for c in range(NCHUNK):
      nxt = c + 1
      if nxt < NCHUNK:
        cps[nxt % 2] = pltpu.async_copy(
            table_hbm.at[idx_v.at[nxt]], bufs[nxt % 2], sems[nxt % 2])
      cps[c % 2].wait()
      pltpu.sync_copy(bufs[c % 2], out_hbm.at[pl.ds(base + c * CH, CH)])

  return k(table_rm, idx3)


def _tc_mlp(emb2, W1b, b1b, W2b, b2b):
  BLK = 2048
  n_rows = emb2.shape[0]

  def body(e_ref, w1_ref, b1_ref, w2_ref, b2_ref, o_ref):
    e = e_ref[...]
    h = jnp.dot(e, w1_ref[...], preferred_element_type=jnp.float32) + b1_ref[...]
    h = h * jax.nn.sigmoid(h)
    o_ref[...] = (
        jnp.dot(h, w2_ref[...], preferred_element_type=jnp.float32) + b2_ref[...]
    )

  return pl.pallas_call(
      body,
      grid=(n_rows // BLK,),
      in_specs=[
          pl.BlockSpec((BLK, 2 * D_IN), lambda i: (i, 0)),
          pl.BlockSpec((2 * D_IN, 2 * D_H), lambda i: (0, 0)),
          pl.BlockSpec((1, 2 * D_H), lambda i: (0, 0)),
          pl.BlockSpec((2 * D_H, 2 * D_H), lambda i: (0, 0)),
          pl.BlockSpec((1, 2 * D_H), lambda i: (0, 0)),
      ],
      out_specs=pl.BlockSpec((BLK, 2 * D_H), lambda i: (i, 0)),
      out_shape=jax.ShapeDtypeStruct((n_rows, 2 * D_H), jnp.float32),
  )(emb2, W1b, b1b, W2b, b2b)


def kernel(t, table, W1, b1, W2, b2):
  Bt, L = t.shape
  packed = _tc_transpose(table.T)
  table_rm = packed.reshape(V_PACK, D_IN)
  tq = 2 * ((t // (2 * CB)) * CB + t % CB) + (t // CB) % 2
  idx3 = tq.T.reshape(NW, NCHUNK, CH)
  emb = _sc_gather(table_rm, idx3)
  emb2 = emb.reshape(B_TOK // 2, 2 * D_IN)
  Z = jnp.zeros_like(W1)
  W1b = jnp.block([[W1, Z], [Z, W1]])
  Zh = jnp.zeros_like(W2)
  W2b = jnp.block([[W2, Zh], [Zh, W2]])
  b1b = jnp.concatenate([b1, b1]).reshape(1, 2 * D_H)
  b2b = jnp.concatenate([b2, b2]).reshape(1, 2 * D_H)
  out2 = _tc_mlp(emb2, W1b, b1b, W2b, b2b)
  return out2.reshape(L, Bt, D_H).transpose(1, 0, 2)


# speedup vs baseline: 1.7907x; 1.0542x over previous
import functools

import jax
import jax.numpy as jnp
from jax import lax
from jax.experimental import pallas as pl
from jax.experimental.pallas import tpu as pltpu
from jax.experimental.pallas import tpu_sc as plsc

D_IN = 64
D_H = 128
B_TOK = 16384 * 20
NW = 32
BPW = B_TOK // NW
CH = 512
NCHUNK = BPW // CH

CB = 8192
SUB = 512
N_SUPER = 62
NP = N_SUPER * CB
V_PACK = 2 * NP
LAST_B_BLK = 122


def _tc_transpose(tableT):
  def body(a_ref, b_ref, o_ref):
    for j in range(CB // SUB):
      sl = pl.ds(j * SUB, SUB)
      o_ref[sl, 0:D_IN] = a_ref[:, sl].T
      o_ref[sl, D_IN:] = b_ref[:, sl].T

  return pl.pallas_call(
      body,
      grid=(N_SUPER,),
      in_specs=[
          pl.BlockSpec((D_IN, CB), lambda i: (0, 2 * i)),
          pl.BlockSpec((D_IN, CB), lambda i: (0, jnp.minimum(2 * i + 1, LAST_B_BLK))),
      ],
      out_specs=pl.BlockSpec((CB, 2 * D_IN), lambda i: (i, 0)),
      out_shape=jax.ShapeDtypeStruct((NP, 2 * D_IN), jnp.float32),
  )(tableT, tableT)


def _sc_gather(table_rm, idx3):
  mesh = plsc.VectorSubcoreMesh(core_axis_name="c", subcore_axis_name="s")

  @functools.partial(
      pl.kernel,
      mesh=mesh,
      compiler_params=pltpu.CompilerParams(use_tc_tiling_on_sc=False),
      out_type=jax.ShapeDtypeStruct((B_TOK, D_IN), jnp.float32),
      scratch_types=[
          pltpu.VMEM((NCHUNK, CH), jnp.int32),
          pltpu.VMEM((CH, D_IN), jnp.float32),
          pltpu.VMEM((CH, D_IN), jnp.float32),
          pltpu.SemaphoreType.DMA,
          pltpu.SemaphoreType.DMA,
      ],
  )
  def k(table_hbm, idx_hbm, out_hbm, idx_v, buf0, buf1, sem0, sem1):
    wid = lax.axis_index("s") * 2 + lax.axis_index("c")
    base = wid * BPW
    pltpu.sync_copy(idx_hbm.at[wid], idx_v)
    bufs = (buf0, buf1)
    sems = (sem0, sem1)
    cps = [None, None]
    cps[0] = pltpu.async_copy(table_hbm.at[idx_v.at[0]], buf0, sem0)
    for c in range(NCHUNK):
      nxt = c + 1
      if nxt < NCHUNK:
        cps[nxt % 2] = pltpu.async_copy(
            table_hbm.at[idx_v.at[nxt]], bufs[nxt % 2], sems[nxt % 2])
      cps[c % 2].wait()
      pltpu.sync_copy(bufs[c % 2], out_hbm.at[pl.ds(base + c * CH, CH)])

  return k(table_rm, idx3)


def _tc_mlp(emb2, W1b, b1b, W2b, b2b):
  BLK = 2048
  n_rows = emb2.shape[0]

  def body(e_ref, w1_ref, b1_ref, w2_ref, b2_ref, o_ref):
    e = e_ref[...]
    h = jnp.dot(e, w1_ref[...], preferred_element_type=jnp.float32) + b1_ref[...]
    h = h * jax.nn.sigmoid(h)
    o_ref[...] = (
        jnp.dot(h, w2_ref[...], preferred_element_type=jnp.float32) + b2_ref[...]
    )

  return pl.pallas_call(
      body,
      grid=(n_rows // BLK,),
      in_specs=[
          pl.BlockSpec((BLK, 2 * D_IN), lambda i: (i, 0)),
          pl.BlockSpec((2 * D_IN, 2 * D_H), lambda i: (0, 0)),
          pl.BlockSpec((1, 2 * D_H), lambda i: (0, 0)),
          pl.BlockSpec((2 * D_H, 2 * D_H), lambda i: (0, 0)),
          pl.BlockSpec((1, 2 * D_H), lambda i: (0, 0)),
      ],
      out_specs=pl.BlockSpec((BLK, 2 * D_H), lambda i: (i, 0)),
      out_shape=jax.ShapeDtypeStruct((n_rows, 2 * D_H), jnp.float32),
  )(emb2, W1b, b1b, W2b, b2b)


def kernel(t, table, W1, b1, W2, b2):
  Bt, L = t.shape
  packed = _tc_transpose(table.T)
  table_rm = packed.reshape(V_PACK, D_IN)
  tq = 2 * ((t // (2 * CB)) * CB + t % CB) + (t // CB) % 2
  idx3 = tq.T.reshape(NW, NCHUNK, CH)
  emb = _sc_gather(table_rm, idx3)
  emb2 = emb.reshape(B_TOK // 2, 2 * D_IN)
  Z = jnp.zeros_like(W1)
  W1b = jnp.block([[W1, Z], [Z, W1]])
  Zh = jnp.zeros_like(W2)
  W2b = jnp.block([[W2, Zh], [Zh, W2]])
  b1b = jnp.concatenate([b1, b1]).reshape(1, 2 * D_H)
  b2b = jnp.concatenate([b2, b2]).reshape(1, 2 * D_H)
  out2 = _tc_mlp(emb2, W1b, b1b, W2b, b2b)
  return out2.reshape(L, Bt, D_H).transpose(1, 0, 2)


# speedup vs baseline: 1.8344x; 1.0244x over previous
import functools

import jax
import jax.numpy as jnp
from jax import lax
from jax.experimental import pallas as pl
from jax.experimental.pallas import tpu as pltpu
from jax.experimental.pallas import tpu_sc as plsc

D_IN = 64
D_H = 128
B_TOK = 16384 * 20
NW = 32
BPW = B_TOK // NW
CH = 512
NCHUNK = BPW // CH

CB = 16384
SUB = 512
N_SUPER = 31
NP = N_SUPER * CB
V_PACK = 2 * NP
LAST_B_BLK = 61


def _tc_transpose(tableT):
  def body(a_ref, b_ref, o_ref):
    for j in range(CB // SUB):
      sl = pl.ds(j * SUB, SUB)
      o_ref[sl, 0:D_IN] = a_ref[:, sl].T
      o_ref[sl, D_IN:] = b_ref[:, sl].T

  return pl.pallas_call(
      body,
      grid=(N_SUPER,),
      in_specs=[
          pl.BlockSpec((D_IN, CB), lambda i: (0, 2 * i)),
          pl.BlockSpec((D_IN, CB), lambda i: (0, jnp.minimum(2 * i + 1, LAST_B_BLK))),
      ],
      out_specs=pl.BlockSpec((CB, 2 * D_IN), lambda i: (i, 0)),
      out_shape=jax.ShapeDtypeStruct((NP, 2 * D_IN), jnp.float32),
  )(tableT, tableT)


def _sc_gather(table_rm, idx3):
  mesh = plsc.VectorSubcoreMesh(core_axis_name="c", subcore_axis_name="s")

  @functools.partial(
      pl.kernel,
      mesh=mesh,
      compiler_params=pltpu.CompilerParams(use_tc_tiling_on_sc=False),
      out_type=jax.ShapeDtypeStruct((B_TOK, D_IN), jnp.float32),
      scratch_types=[
          pltpu.VMEM((NCHUNK, CH), jnp.int32),
          pltpu.VMEM((CH, D_IN), jnp.float32),
          pltpu.VMEM((CH, D_IN), jnp.float32),
          pltpu.SemaphoreType.DMA,
          pltpu.SemaphoreType.DMA,
      ],
  )
  def k(table_hbm, idx_hbm, out_hbm, idx_v, buf0, buf1, sem0, sem1):
    wid = lax.axis_index("s") * 2 + lax.axis_index("c")
    base = wid * BPW
    pltpu.sync_copy(idx_hbm.at[wid], idx_v)
    bufs = (buf0, buf1)
    sems = (sem0, sem1)
    cps = [None, None]
    cps[0] = pltpu.async_copy(table_hbm.at[idx_v.at[0]], buf0, sem0)
    for c in range(NCHUNK):
      nxt = c + 1
      if nxt < NCHUNK:
        cps[nxt % 2] = pltpu.async_copy(
            table_hbm.at[idx_v.at[nxt]], bufs[nxt % 2], sems[nxt % 2])
      cps[c % 2].wait()
      pltpu.sync_copy(bufs[c % 2], out_hbm.at[pl.ds(base + c * CH, CH)])

  return k(table_rm, idx3)


def _tc_mlp(emb2, W1b, b1b, W2b, b2b):
  BLK = 2048
  n_rows = emb2.shape[0]

  def body(e_ref, w1_ref, b1_ref, w2_ref, b2_ref, o_ref):
    e = e_ref[...]
    h = jnp.dot(e, w1_ref[...], preferred_element_type=jnp.float32) + b1_ref[...]
    h = h * jax.nn.sigmoid(h)
    o_ref[...] = (
        jnp.dot(h, w2_ref[...], preferred_element_type=jnp.float32) + b2_ref[...]
    )

  return pl.pallas_call(
      body,
      grid=(n_rows // BLK,),
      in_specs=[
          pl.BlockSpec((BLK, 2 * D_IN), lambda i: (i, 0)),
          pl.BlockSpec((2 * D_IN, 2 * D_H), lambda i: (0, 0)),
          pl.BlockSpec((1, 2 * D_H), lambda i: (0, 0)),
          pl.BlockSpec((2 * D_H, 2 * D_H), lambda i: (0, 0)),
          pl.BlockSpec((1, 2 * D_H), lambda i: (0, 0)),
      ],
      out_specs=pl.BlockSpec((BLK, 2 * D_H), lambda i: (i, 0)),
      out_shape=jax.ShapeDtypeStruct((n_rows, 2 * D_H), jnp.float32),
  )(emb2, W1b, b1b, W2b, b2b)


def kernel(t, table, W1, b1, W2, b2):
  Bt, L = t.shape
  packed = _tc_transpose(table.T)
  table_rm = packed.reshape(V_PACK, D_IN)
  tq = 2 * ((t // (2 * CB)) * CB + t % CB) + (t // CB) % 2
  idx3 = tq.T.reshape(NW, NCHUNK, CH)
  emb = _sc_gather(table_rm, idx3)
  emb2 = emb.reshape(B_TOK // 2, 2 * D_IN)
  Z = jnp.zeros_like(W1)
  W1b = jnp.block([[W1, Z], [Z, W1]])
  Zh = jnp.zeros_like(W2)
  W2b = jnp.block([[W2, Zh], [Zh, W2]])
  b1b = jnp.concatenate([b1, b1]).reshape(1, 2 * D_H)
  b2b = jnp.concatenate([b2, b2]).reshape(1, 2 * D_H)
  out2 = _tc_mlp(emb2, W1b, b1b, W2b, b2b)
  return out2.reshape(L, Bt, D_H).transpose(1, 0, 2)
